# Initial kernel scaffold; baseline (speedup 1.0000x reference)
import jax
import jax.numpy as jnp
from jax.experimental import pallas as pl


def kernel(x, edge_index, edge_attr, n_nodes, We1, be1, W11, b11, W12, b12, g1, bt1, We2, be2, W21, b21, W22, b22, g2, bt2, We3, be3, W31, b31, W32, b32, g3, bt3, Wf1, bf1, Wf2, bf2):
    raise NotImplementedError("write your pallas kernel here")


# speedup vs baseline: 2.8657x; 2.8657x over previous
import functools

import jax
import jax.numpy as jnp
from jax import lax
from jax.experimental import pallas as pl
from jax.experimental.pallas import tpu as pltpu
from jax.experimental.pallas import tpu_sc as plsc

N = 10000
E = 320000
NC, NS, LANES = 2, 16, 16
NW = NC * NS

K = 80
SPAN = 624
TAIL = N - NS * SPAN
ZR = 104


def _mk_sc_agg(chan_split):
    epw = E // NS if chan_split else E // NW
    nchunks = epw // K

    mesh = plsc.VectorSubcoreMesh(
        core_axis_name="c", subcore_axis_name="s",
        num_cores=NC, num_subcores=NS)

    @functools.partial(
        pl.kernel,
        out_type=jax.ShapeDtypeStruct((NC, N, 128), jnp.float32),
        mesh=mesh,
        scratch_types=[
            pltpu.VMEM((K,), jnp.int32),
            pltpu.VMEM((K,), jnp.int32),
            pltpu.VMEM((K, 128), jnp.float32),
            pltpu.VMEM((K, 128), jnp.float32),
            pltpu.VMEM((ZR, 128), jnp.float32),
            pltpu.VMEM_SHARED((N, 128), jnp.float32),
            pltpu.SemaphoreType.DMA,
            pltpu.SemaphoreType.DMA,
        ],
    )
    def k(x_hbm, e_hbm, src_hbm, dst_hbm, out_hbm,
          src_v, dst_v, rows_v, e_v, zero_v, agg_sh, sem1, sem2):
        c = lax.axis_index("c")
        s = lax.axis_index("s")

        def zrow(r, _):
            for l in range(8):
                zero_v[r, pl.ds(l * 16, 16)] = jnp.zeros((16,), jnp.float32)
            return 0
        lax.fori_loop(0, ZR, zrow, 0)
        for j in range(SPAN // ZR):
            pltpu.sync_copy(zero_v.at[pl.ds(0, ZR)],
                            agg_sh.at[pl.ds(s * SPAN + j * ZR, ZR)])

        @pl.when(s == 0)
        def _():
            pltpu.sync_copy(zero_v.at[pl.ds(0, TAIL)],
                            agg_sh.at[pl.ds(NS * SPAN, TAIL)])
        plsc.subcore_barrier()

        if chan_split:
            ebase0 = s * epw
            off = c * E
        else:
            ebase0 = (s * NC + c) * epw
            off = 0

        def chunk(i, _):
            base = ebase0 + i * K
            cp1 = pltpu.async_copy(src_hbm.at[pl.ds(off + base, K)], src_v, sem1)
            cp2 = pltpu.async_copy(dst_hbm.at[pl.ds(base, K)], dst_v, sem2)
            cp1.wait()
            cp2.wait()
            g = pltpu.async_copy(x_hbm.at[src_v], rows_v, sem1)
            ce = pltpu.async_copy(e_hbm.at[pl.ds(off + base, K)], e_v, sem2)
            g.wait()
            ce.wait()

            def row(r, _):
                for l in range(8):
                    sl = pl.ds(l * 16, 16)
                    rows_v[r, sl] = jnp.maximum(rows_v[r, sl] + e_v[r, sl], 0.0)
                return 0
            lax.fori_loop(0, K, row, 0)
            pltpu.sync_copy(rows_v, agg_sh.at[dst_v], add=True)
            return 0
        lax.fori_loop(0, nchunks, chunk, 0)
        plsc.subcore_barrier()
        pltpu.sync_copy(agg_sh.at[pl.ds(s * SPAN, SPAN)],
                        out_hbm.at[c, pl.ds(s * SPAN, SPAN)])

        @pl.when(s == 0)
        def _():
            pltpu.sync_copy(agg_sh.at[pl.ds(NS * SPAN, TAIL)],
                            out_hbm.at[c, pl.ds(NS * SPAN, TAIL)])

    return k


_sc_agg_edge_split = _mk_sc_agg(False)
_sc_agg_chan_split = _mk_sc_agg(True)


BE = 4000
BN = 1000


def _edge_lin_1out(ea_ref, w_ref, b_ref, o_ref):
    o_ref[...] = jnp.dot(ea_ref[...], w_ref[...],
                         preferred_element_type=jnp.float32) + b_ref[...]


def _edge_lin_split(ea_ref, w_ref, b_ref, o_ref):
    e = jnp.dot(ea_ref[...], w_ref[...],
                preferred_element_type=jnp.float32) + b_ref[...]
    o_ref[0] = e[:, :128]
    o_ref[1] = e[:, 128:]


def _edge_linear(edge_attr, w, b, split):
    cout = w.shape[1]
    if split:
        out_shape = jax.ShapeDtypeStruct((2, E, 128), jnp.float32)
        out_spec = pl.BlockSpec((2, BE, 128), lambda i: (0, i, 0))
        body = _edge_lin_split
    else:
        out_shape = jax.ShapeDtypeStruct((E, cout), jnp.float32)
        out_spec = pl.BlockSpec((BE, cout), lambda i: (i, 0))
        body = _edge_lin_1out
    return pl.pallas_call(
        body,
        grid=(E // BE,),
        in_specs=[
            pl.BlockSpec((BE, 16), lambda i: (i, 0)),
            pl.BlockSpec(w.shape, lambda i: (0, 0)),
            pl.BlockSpec((1, cout), lambda i: (0, 0)),
        ],
        out_specs=out_spec,
        out_shape=out_shape,
    )(edge_attr, w, b.reshape(1, cout))


def _mlp_layer(x_parts, agg, w1, b1, w2, b2, in_split, out_split):
    cin = w1.shape[0]
    chid = w1.shape[1]
    cout = w2.shape[1]
    nb = N // BN

    def body(x_ref, a_ref, w1_ref, b1_ref, w2_ref, b2_ref,
             z_ref, s_ref, q_ref):
        if in_split:
            h = jnp.concatenate([x_ref[0] + a_ref[0], x_ref[1] + a_ref[1]],
                                axis=-1)
        else:
            h = x_ref[...] + a_ref[0] + a_ref[1]
        t = jnp.maximum(jnp.dot(h, w1_ref[...],
                                preferred_element_type=jnp.float32)
                        + b1_ref[...], 0.0)
        z = jnp.maximum(jnp.dot(t, w2_ref[...],
                                preferred_element_type=jnp.float32)
                        + b2_ref[...], 0.0)
        if out_split:
            z_ref[0] = z[:, :128]
            z_ref[1] = z[:, 128:]
        else:
            z_ref[...] = z
        s_ref[0] = jnp.sum(z, axis=0, keepdims=True)
        q_ref[0] = jnp.sum(z * z, axis=0, keepdims=True)

    if in_split:
        x_spec = pl.BlockSpec((2, BN, 128), lambda i: (0, i, 0))
    else:
        x_spec = pl.BlockSpec((BN, cin), lambda i: (i, 0))
    if out_split:
        z_spec = pl.BlockSpec((2, BN, 128), lambda i: (0, i, 0))
        z_shape = jax.ShapeDtypeStruct((2, N, 128), jnp.float32)
    else:
        z_spec = pl.BlockSpec((BN, cout), lambda i: (i, 0))
        z_shape = jax.ShapeDtypeStruct((N, cout), jnp.float32)

    return pl.pallas_call(
        body,
        grid=(nb,),
        in_specs=[
            x_spec,
            pl.BlockSpec((2, BN, 128), lambda i: (0, i, 0)),
            pl.BlockSpec((cin, chid), lambda i: (0, 0)),
            pl.BlockSpec((1, chid), lambda i: (0, 0)),
            pl.BlockSpec((chid, cout), lambda i: (0, 0)),
            pl.BlockSpec((1, cout), lambda i: (0, 0)),
        ],
        out_specs=[
            z_spec,
            pl.BlockSpec((1, 1, cout), lambda i: (i, 0, 0)),
            pl.BlockSpec((1, 1, cout), lambda i: (i, 0, 0)),
        ],
        out_shape=[
            z_shape,
            jax.ShapeDtypeStruct((nb, 1, cout), jnp.float32),
            jax.ShapeDtypeStruct((nb, 1, cout), jnp.float32),
        ],
    )(x_parts, agg, w1, b1.reshape(1, chid), w2, b2.reshape(1, cout))


def _bn_norm(z, s, q, g, bt, in_split, out_split):
    cout = g.shape[0]
    nb = N // BN

    def body(z_ref, s_ref, q_ref, g_ref, bt_ref, o_ref):
        m = jnp.sum(s_ref[:, 0, :], axis=0, keepdims=True) / N
        v = jnp.sum(q_ref[:, 0, :], axis=0, keepdims=True) / N - m * m
        scale = g_ref[...] * lax.rsqrt(v + 1e-5)
        shift = bt_ref[...] - m * scale
        if in_split:
            z = jnp.concatenate([z_ref[0], z_ref[1]], axis=-1)
        else:
            z = z_ref[...]
        h = z * scale + shift
        if out_split:
            o_ref[0] = h[:, :128]
            o_ref[1] = h[:, 128:]
        else:
            o_ref[...] = h

    if in_split:
        z_spec = pl.BlockSpec((2, BN, 128), lambda i: (0, i, 0))
    else:
        z_spec = pl.BlockSpec((BN, cout), lambda i: (i, 0))
    if out_split:
        o_spec = pl.BlockSpec((2, BN, 128), lambda i: (0, i, 0))
        o_shape = jax.ShapeDtypeStruct((2, N, 128), jnp.float32)
    else:
        o_spec = pl.BlockSpec((BN, cout), lambda i: (i, 0))
        o_shape = jax.ShapeDtypeStruct((N, cout), jnp.float32)

    return pl.pallas_call(
        body,
        grid=(nb,),
        in_specs=[
            z_spec,
            pl.BlockSpec((nb, 1, cout), lambda i: (0, 0, 0)),
            pl.BlockSpec((nb, 1, cout), lambda i: (0, 0, 0)),
            pl.BlockSpec((1, cout), lambda i: (0, 0)),
            pl.BlockSpec((1, cout), lambda i: (0, 0)),
        ],
        out_specs=o_spec,
        out_shape=o_shape,
    )(z, s, q, g.reshape(1, cout), bt.reshape(1, cout))


def _readout(master, wf1, bf1, wf2, bf2):
    b = master.shape[0]

    def body(m_ref, w1_ref, b1_ref, w2_ref, b2_ref, o_ref):
        t = jnp.maximum(jnp.dot(m_ref[...], w1_ref[...],
                                preferred_element_type=jnp.float32)
                        + b1_ref[...], 0.0)
        o_ref[...] = jnp.dot(t, w2_ref[...],
                             preferred_element_type=jnp.float32) + b2_ref[...]

    return pl.pallas_call(
        body,
        out_shape=jax.ShapeDtypeStruct((b, 1), jnp.float32),
    )(master, wf1, bf1.reshape(1, 16), wf2, bf2.reshape(1, 1))


def kernel(x, edge_index, edge_attr, n_nodes,
           We1, be1, W11, b11, W12, b12, g1, bt1,
           We2, be2, W21, b21, W22, b22, g2, bt2,
           We3, be3, W31, b31, W32, b32, g3, bt3,
           Wf1, bf1, Wf2, bf2):
    src = edge_index[0].astype(jnp.int32)
    dst = edge_index[1].astype(jnp.int32)
    src2 = jnp.concatenate([src, src + N])

    e1 = _edge_linear(edge_attr, We1, be1, split=False)
    agg1 = _sc_agg_edge_split(x, e1, src, dst)
    e2 = _edge_linear(edge_attr, We2, be2, split=True)
    e3 = _edge_linear(edge_attr, We3, be3, split=False)

    z1, s1, q1 = _mlp_layer(x, agg1, W11, b11, W12, b12,
                            in_split=False, out_split=False)
    h1 = _bn_norm(z1, s1, q1, g1, bt1, in_split=False, out_split=True)

    agg2 = _sc_agg_chan_split(h1.reshape(2 * N, 128), e2.reshape(2 * E, 128),
                              src2, dst)
    z2, s2, q2 = _mlp_layer(h1, agg2, W21, b21, W22, b22,
                            in_split=True, out_split=False)
    h2 = _bn_norm(z2, s2, q2, g2, bt2, in_split=False, out_split=False)

    agg3 = _sc_agg_edge_split(h2, e3, src, dst)
    z3, s3, q3 = _mlp_layer(h2, agg3, W31, b31, W32, b32,
                            in_split=False, out_split=False)
    h3 = _bn_norm(z3, s3, q3, g3, bt3, in_split=False, out_split=False)

    last = jnp.cumsum(n_nodes) - 1
    master = h3[last]
    return _readout(master, Wf1, bf1, Wf2, bf2)
